# baseline (speedup 1.0000x reference)
import functools

import jax
import jax.numpy as jnp
from jax import lax
from jax.experimental import pallas as pl
from jax.experimental.pallas import tpu as pltpu
from jax.experimental.pallas import tpu_sc as plsc

_NUM_CORES = 2
_NUM_SUBCORES = 16
_NUM_WORKERS = _NUM_CORES * _NUM_SUBCORES
_CHUNK = 80
_K = 5
_LH = 2
_PADW = 128


def kernel(x, timestamps, time_embedding):
    batch, seq, d = x.shape
    n = batch * seq
    xf = x.reshape(n // 8, 8, d)
    idx = timestamps.reshape(n).astype(jnp.int32)
    tab_pad = jnp.pad(time_embedding, ((0, 0), (0, _PADW - d)))

    rows_per_w = n // _NUM_WORKERS
    n_chunks = rows_per_w // _CHUNK
    n_groups = n_chunks // _K
    assert n % _NUM_WORKERS == 0
    assert rows_per_w % _CHUNK == 0 and n_chunks % _K == 0 and n_groups >= 3

    mesh = plsc.VectorSubcoreMesh(core_axis_name="c", subcore_axis_name="s")

    @functools.partial(
        pl.kernel,
        out_type=jax.ShapeDtypeStruct((n // 8, 8, d), jnp.float32),
        mesh=mesh,
        scratch_types=[
            pltpu.VMEM((_K, _CHUNK), jnp.int32),
            pltpu.VMEM((_K, _CHUNK, _PADW), jnp.float32),
            pltpu.VMEM((_K, _CHUNK // 8, 8, 64), jnp.float32),
            pltpu.SemaphoreType.DMA((_K,)),
            pltpu.SemaphoreType.DMA((_K,)),
            pltpu.SemaphoreType.DMA((_K,)),
            pltpu.SemaphoreType.DMA((_K,)),
        ],
    )
    def sc_kernel(x_hbm, idx_hbm, tab_hbm, out_hbm,
                  idx_v, g_v, x_v, sem_i, sem_x, sem_g, sem_s):
        wid = lax.axis_index("s") * _NUM_CORES + lax.axis_index("c")
        base = wid * rows_per_w

        def off(i):
            return base + i * _CHUNK

        def issue_loads(i, b):
            pltpu.async_copy(idx_hbm.at[pl.ds(off(i), _CHUNK)],
                             idx_v.at[b], sem_i.at[b])
            pltpu.async_copy(x_hbm.at[pl.ds(off(i) // 8, _CHUNK // 8)],
                             x_v.at[b], sem_x.at[b])

        def wait_loads(i, b):
            pltpu.make_async_copy(idx_hbm.at[pl.ds(off(i), _CHUNK)],
                                  idx_v.at[b], sem_i.at[b]).wait()
            pltpu.make_async_copy(x_hbm.at[pl.ds(off(i) // 8, _CHUNK // 8)],
                                  x_v.at[b], sem_x.at[b]).wait()

        def issue_gather(b):
            pltpu.async_copy(tab_hbm.at[idx_v.at[b]], g_v.at[b],
                             sem_g.at[b])

        def wait_gather(b):
            pltpu.make_async_copy(tab_hbm.at[idx_v.at[b]], g_v.at[b],
                                  sem_g.at[b]).wait()

        def issue_store(i, b):
            pltpu.async_copy(x_v.at[b],
                             out_hbm.at[pl.ds(off(i) // 8, _CHUNK // 8)],
                             sem_s.at[b])

        def wait_store(i, b):
            pltpu.make_async_copy(x_v.at[b],
                                  out_hbm.at[pl.ds(off(i) // 8, _CHUNK // 8)],
                                  sem_s.at[b]).wait()

        def add_chunk(b):
            @pl.loop(0, _CHUNK // 8)
            def _(q):
                for rr in range(8):
                    for j in range(d // 16):
                        sl = pl.ds(j * 16, 16)
                        x_v[b, q, rr, sl] = (x_v[b, q, rr, sl]
                                             + g_v[b, q * 8 + rr, sl])

        def slot(i, b, first=False, warm=False, tail=False):
            if not first:
                pb = (b - 1) % _K
                wait_gather(pb)
                add_chunk(pb)
                issue_store(i - 1, pb)
            wait_loads(i, b)
            issue_gather(b)
            if not tail:
                wb = (b + _LH) % _K
                if warm:
                    wait_store(i - (_K - _LH), wb)
                issue_loads(i + _LH, wb)

        issue_loads(0, 0)
        issue_loads(1, 1)
        for b in range(_K):
            slot(b, b, first=(b == 0), warm=(b >= _K - _LH))

        @pl.loop(1, n_groups - 1)
        def _(g):
            i0 = g * _K
            for b in range(_K):
                slot(i0 + b, b, warm=True)

        last0 = (n_groups - 1) * _K
        for b in range(_K):
            i = last0 + b
            slot(i, b, warm=True, tail=(i + _LH >= n_chunks))

        wait_gather((_K - 1) % _K)
        add_chunk((_K - 1) % _K)
        issue_store(n_chunks - 1, (_K - 1) % _K)
        for b in range(_K):
            wait_store(n_chunks - _K + b, b)

    out = sc_kernel(xf, idx, tab_pad)
    return out.reshape(batch, seq, d)


# speedup vs baseline: 1.1864x; 1.1864x over previous
import functools

import jax
import jax.numpy as jnp
from jax import lax
from jax.experimental import pallas as pl
from jax.experimental.pallas import tpu as pltpu
from jax.experimental.pallas import tpu_sc as plsc

_NUM_CORES = 2
_NUM_SUBCORES = 16
_NUM_WORKERS = _NUM_CORES * _NUM_SUBCORES
_CHUNK = 80
_K = 5
_LH = 2
_PADW = 128


def kernel(x, timestamps, time_embedding):
    batch, seq, d = x.shape
    n = batch * seq
    xf = x.reshape(n // 8, 8, d)
    idx = timestamps.reshape(n).astype(jnp.int32)
    tab_pad = jnp.pad(time_embedding, ((0, 0), (0, _PADW - d)))

    rows_per_w = n // _NUM_WORKERS
    n_chunks = rows_per_w // _CHUNK
    n_groups = n_chunks // _K
    assert n % _NUM_WORKERS == 0
    assert rows_per_w % _CHUNK == 0 and n_chunks % _K == 0 and n_groups >= 3

    mesh = plsc.VectorSubcoreMesh(core_axis_name="c", subcore_axis_name="s")

    @functools.partial(
        pl.kernel,
        out_type=jax.ShapeDtypeStruct((n // 8, 8, d), jnp.float32),
        mesh=mesh,
        scratch_types=[
            pltpu.VMEM((_K, _CHUNK), jnp.int32),
            pltpu.VMEM((_K, _CHUNK, _PADW), jnp.float32),
            pltpu.VMEM((_K, _CHUNK // 8, 8, 64), jnp.float32),
            pltpu.SemaphoreType.DMA((_K,)),
            pltpu.SemaphoreType.DMA((_K,)),
            pltpu.SemaphoreType.DMA((_K,)),
            pltpu.SemaphoreType.DMA((_K,)),
        ],
    )
    def sc_kernel(x_hbm, idx_hbm, tab_hbm, out_hbm,
                  idx_v, g_v, x_v, sem_i, sem_x, sem_g, sem_s):
        wid = lax.axis_index("s") * _NUM_CORES + lax.axis_index("c")
        base = wid * rows_per_w

        def off(i):
            return base + i * _CHUNK

        def issue_loads(i, b):
            pltpu.async_copy(idx_hbm.at[pl.ds(off(i), _CHUNK)],
                             idx_v.at[b], sem_i.at[b])
            pltpu.async_copy(x_hbm.at[pl.ds(off(i) // 8, _CHUNK // 8)],
                             x_v.at[b], sem_x.at[b])

        def wait_loads(i, b):
            pltpu.make_async_copy(idx_hbm.at[pl.ds(off(i), _CHUNK)],
                                  idx_v.at[b], sem_i.at[b]).wait()
            pltpu.make_async_copy(x_hbm.at[pl.ds(off(i) // 8, _CHUNK // 8)],
                                  x_v.at[b], sem_x.at[b]).wait()

        def issue_gather(b):
            pltpu.async_copy(tab_hbm.at[idx_v.at[b]], g_v.at[b],
                             sem_g.at[b])

        def wait_gather(b):
            pltpu.make_async_copy(tab_hbm.at[idx_v.at[b]], g_v.at[b],
                                  sem_g.at[b]).wait()

        def issue_store(i, b):
            pltpu.async_copy(x_v.at[b],
                             out_hbm.at[pl.ds(off(i) // 8, _CHUNK // 8)],
                             sem_s.at[b])

        def wait_store(i, b):
            pltpu.make_async_copy(x_v.at[b],
                                  out_hbm.at[pl.ds(off(i) // 8, _CHUNK // 8)],
                                  sem_s.at[b]).wait()

        def add_chunk(b):
            @pl.loop(0, _CHUNK // 8)
            def _(q):
                for rr in range(8):
                    for j in range(d // 16):
                        sl = pl.ds(j * 16, 16)
                        x_v[b, q, rr, sl] = (x_v[b, q, rr, sl]
                                             + g_v[b, q * 8 + rr, sl])

        def slot(i, b, first=False, warm=False, tail=False):
            wait_loads(i, b)
            issue_gather(b)
            if not tail:
                wb = (b + _LH) % _K
                if warm:
                    wait_store(i - (_K - _LH), wb)
                issue_loads(i + _LH, wb)
            if not first:
                pb = (b - 1) % _K
                wait_gather(pb)
                add_chunk(pb)
                issue_store(i - 1, pb)

        issue_loads(0, 0)
        issue_loads(1, 1)
        for b in range(_K):
            slot(b, b, first=(b == 0), warm=(b >= _K - _LH))

        @pl.loop(1, n_groups - 1)
        def _(g):
            i0 = g * _K
            for b in range(_K):
                slot(i0 + b, b, warm=True)

        last0 = (n_groups - 1) * _K
        for b in range(_K):
            i = last0 + b
            slot(i, b, warm=True, tail=(i + _LH >= n_chunks))

        wait_gather((_K - 1) % _K)
        add_chunk((_K - 1) % _K)
        issue_store(n_chunks - 1, (_K - 1) % _K)
        for b in range(_K):
            wait_store(n_chunks - _K + b, b)

    out = sc_kernel(xf, idx, tab_pad)
    return out.reshape(batch, seq, d)
